# Initial kernel scaffold; baseline (speedup 1.0000x reference)
import jax
import jax.numpy as jnp
from jax.experimental import pallas as pl


def kernel(features, adj_edge_index, adj_edge_weight, weight):
    raise NotImplementedError("write your pallas kernel here")


# speedup vs baseline: 2.6201x; 2.6201x over previous
import jax
import jax.numpy as jnp
from jax import lax
from jax.experimental import pallas as pl
from jax.experimental.pallas import tpu as pltpu
from jax.experimental.pallas import tpu_sc as plsc

_N = 10000
_E = 320000
_D = 128
_NW = 32
_K = 128
_NCHUNK = 80
_EPW = _K * _NCHUNK
_EPAD = _EPW * _NW
_RPS = _N // 16
_ZROWS = 125


def _tanh_matmul(x, w):
    blk = 1000

    def body(x_ref, w_ref, o_ref):
        o_ref[...] = jnp.tanh(
            jnp.dot(x_ref[...], w_ref[...], preferred_element_type=jnp.float32))

    return pl.pallas_call(
        body,
        grid=(_N // blk,),
        in_specs=[pl.BlockSpec((blk, _D), lambda i: (i, 0)),
                  pl.BlockSpec((_D, _D), lambda i: (0, 0))],
        out_specs=pl.BlockSpec((blk, _D), lambda i: (i, 0)),
        out_shape=jax.ShapeDtypeStruct((_N, _D), jnp.float32),
    )(x, w)


def _add2(a, b):
    blk = 1000

    def body(a_ref, b_ref, o_ref):
        o_ref[...] = a_ref[...] + b_ref[...]

    return pl.pallas_call(
        body,
        grid=(_N // blk,),
        in_specs=[pl.BlockSpec((blk, _D), lambda i: (i, 0)),
                  pl.BlockSpec((blk, _D), lambda i: (i, 0))],
        out_specs=pl.BlockSpec((blk, _D), lambda i: (i, 0)),
        out_shape=jax.ShapeDtypeStruct((_N, _D), jnp.float32),
    )(a, b)


def _spmm_body(h3, cols3, rows3, w3, out, acc, col_v, row_v, w_v, gath):
    c = lax.axis_index("c")
    s = lax.axis_index("s")
    wid = c * 16 + s

    pltpu.sync_copy(cols3.at[wid], col_v)
    pltpu.sync_copy(rows3.at[wid], row_v)
    pltpu.sync_copy(w3.at[wid], w_v)

    zv = jnp.zeros((16,), jnp.float32)

    @pl.loop(0, _ZROWS)
    def _(t):
        for j in range(8):
            gath[t, 0, pl.ds(16 * j, 16)] = zv

    base = s * _RPS
    for t in range(_RPS // _ZROWS):
        pltpu.sync_copy(gath.at[pl.ds(0, _ZROWS)],
                        acc.at[pl.ds(base + t * _ZROWS, _ZROWS)])
    plsc.subcore_barrier()

    @pl.loop(0, _NCHUNK)
    def _(i):
        pltpu.sync_copy(h3.at[col_v.at[i]], gath)

        @pl.loop(0, _K // 16)
        def _(g):
            w16 = w_v[i, pl.ds(16 * g, 16)]
            for l in range(16):
                wv = w16.at[jnp.full((16,), l, jnp.int32)].get(
                    mode="promise_in_bounds")
                for j in range(8):
                    sl = (16 * g + l, 0, pl.ds(16 * j, 16))
                    gath[sl] = gath[sl] * wv

        pltpu.sync_copy(gath, acc.at[row_v.at[i]], add=True)

    plsc.subcore_barrier()
    pltpu.sync_copy(acc.at[pl.ds(base, _RPS)],
                    out.at[c, pl.ds(base, _RPS)])


def _spmm(h3, cols3, rows3, w3):
    mesh = plsc.VectorSubcoreMesh(core_axis_name="c", subcore_axis_name="s")
    f = pl.kernel(
        _spmm_body,
        out_type=jax.ShapeDtypeStruct((2, _N, 1, _D), jnp.float32),
        mesh=mesh,
        scratch_types=[
            pltpu.VMEM_SHARED((_N, 1, _D), jnp.float32),
            pltpu.VMEM((_NCHUNK, _K), jnp.int32),
            pltpu.VMEM((_NCHUNK, _K), jnp.int32),
            pltpu.VMEM((_NCHUNK, _K), jnp.float32),
            pltpu.VMEM((_K, 1, _D), jnp.float32),
        ],
    )
    return f(h3, cols3, rows3, w3)


def kernel(features, adj_edge_index, adj_edge_weight, weight):
    h = _tanh_matmul(features, weight)
    rows = adj_edge_index[0]
    cols = adj_edge_index[1]
    pad = _EPAD - _E
    cols3 = jnp.pad(cols, (0, pad)).reshape(_NW, _NCHUNK, _K)
    rows3 = jnp.pad(rows, (0, pad)).reshape(_NW, _NCHUNK, _K)
    w3 = jnp.pad(adj_edge_weight, (0, pad)).reshape(_NW, _NCHUNK, _K)
    out = h
    for _ in range(2):
        p = _spmm(out.reshape(_N, 1, _D), cols3, rows3, w3)
        out = _add2(p[0, :, 0, :], p[1, :, 0, :])
    return out
